# Initial kernel scaffold; baseline (speedup 1.0000x reference)
import jax
import jax.numpy as jnp
from jax.experimental import pallas as pl


def kernel(cat_tensor, W_bus_id, W_station_id, W_time_period, W_direction):
    raise NotImplementedError("write your pallas kernel here")


# speedup vs baseline: 2.3255x; 2.3255x over previous
import functools

import jax
import jax.numpy as jnp
from jax import lax
from jax.experimental import pallas as pl
from jax.experimental.pallas import tpu as pltpu
from jax.experimental.pallas import tpu_sc as plsc

B = 16384
D_OUT = 151
NC, NS, NL = 2, 16, 16
NW = NC * NS
B_PER_W = B // NW
N_BATCH = 2
B_PER_BATCH = B_PER_W // N_BATCH
ACC_WORDS = B_PER_BATCH * D_OUT


def _body(idx_hbm, wb_hbm, ws_hbm, wt_hbm, wd_hbm, out_hbm,
          idx0_v, idx1_v, idx2_v, idx3_v, dir_v,
          bus_v, sta_v, tim_v, acc_v,
          sem0, sem1, sem2):
    wid = lax.axis_index("s") * NC + lax.axis_index("c")
    base = wid * B_PER_W

    pltpu.sync_copy(idx_hbm.at[0, pl.ds(base, B_PER_W)], idx0_v)
    pltpu.sync_copy(idx_hbm.at[1, pl.ds(base, B_PER_W)], idx1_v)
    pltpu.sync_copy(idx_hbm.at[2, pl.ds(base, B_PER_W)], idx2_v)
    pltpu.sync_copy(idx_hbm.at[3, pl.ds(base, B_PER_W)], idx3_v)
    pltpu.sync_copy(wd_hbm, dir_v)

    cp0 = pltpu.async_copy(wb_hbm.at[idx0_v], bus_v, sem0)
    cp1 = pltpu.async_copy(ws_hbm.at[idx1_v], sta_v, sem1)
    cp2 = pltpu.async_copy(wt_hbm.at[idx2_v], tim_v, sem2)
    cp0.wait()
    cp1.wait()
    cp2.wait()

    iota = lax.iota(jnp.int32, NL)
    cols = [iota, iota + 16, iota + 32, iota + 34]
    chunk_offs = (0, 16, 32, 34)

    for batch in range(N_BATCH):
        row0 = batch * B_PER_BATCH

        def repack_row(b, _):
            rowv = jnp.full((NL,), row0 + b, jnp.int32)
            dbase = jnp.full((NL,), b * D_OUT, jnp.int32) + iota
            for k, buf in enumerate((bus_v, sta_v, tim_v)):
                for c_i, c in enumerate(chunk_offs):
                    v = plsc.load_gather(buf, [rowv, cols[c_i]])
                    plsc.store_scatter(acc_v, [dbase + (k * 50 + c)], v)
            return 0

        lax.fori_loop(0, B_PER_BATCH, repack_row, 0)

        for j in range(B_PER_BATCH // NL):
            dvals = plsc.load_gather(
                dir_v, [idx3_v[pl.ds(row0 + j * NL, NL)]])
            dst = (j * NL + iota) * D_OUT + 150
            plsc.store_scatter(acc_v, [dst], dvals)

        pltpu.sync_copy(
            acc_v,
            out_hbm.at[pl.ds((base + row0) * D_OUT, ACC_WORDS)])


@jax.jit
def _run(idx_t, wb, ws, wt, dir16):
    mesh = plsc.VectorSubcoreMesh(core_axis_name="c", subcore_axis_name="s")
    out_flat = pl.kernel(
        _body,
        out_type=jax.ShapeDtypeStruct((B * D_OUT,), jnp.float32),
        mesh=mesh,
        scratch_types=[
            pltpu.VMEM((B_PER_W,), jnp.int32),
            pltpu.VMEM((B_PER_W,), jnp.int32),
            pltpu.VMEM((B_PER_W,), jnp.int32),
            pltpu.VMEM((B_PER_W,), jnp.int32),
            pltpu.VMEM((NL,), jnp.float32),
            pltpu.VMEM((B_PER_W, 50), jnp.float32),
            pltpu.VMEM((B_PER_W, 50), jnp.float32),
            pltpu.VMEM((B_PER_W, 50), jnp.float32),
            pltpu.VMEM((ACC_WORDS,), jnp.float32),
            pltpu.SemaphoreType.DMA,
            pltpu.SemaphoreType.DMA,
            pltpu.SemaphoreType.DMA,
        ],
        compiler_params=pltpu.CompilerParams(
            use_tc_tiling_on_sc=False, needs_layout_passes=False),
    )(idx_t, wb, ws, wt, dir16)
    return out_flat.reshape(B, D_OUT)


def kernel(cat_tensor, W_bus_id, W_station_id, W_time_period, W_direction):
    idx_t = cat_tensor.T.astype(jnp.int32)
    dir16 = jnp.pad(W_direction[:, 0], (0, NL - W_direction.shape[0]))
    return _run(idx_t, W_bus_id, W_station_id, W_time_period, dir16)


# speedup vs baseline: 2.4403x; 1.0494x over previous
import functools

import jax
import jax.numpy as jnp
from jax import lax
from jax.experimental import pallas as pl
from jax.experimental.pallas import tpu as pltpu
from jax.experimental.pallas import tpu_sc as plsc

B = 16384
D_OUT = 151
NC, NS, NL = 2, 16, 16
NW = NC * NS
B_PER_W = B // NW
HALF = B_PER_W // 2
ACC_WORDS = B_PER_W * D_OUT
ROW_OFF = (0, 128, 384)


def _body(idx_hbm, wb_hbm, ws_hbm, wt_hbm, wd_hbm, out_hbm,
          idx0_v, idx1_v, idx2_v, idx3_v, dir_v, tab_v, acc_v,
          sem0, sem1):
    wid = lax.axis_index("s") * NC + lax.axis_index("c")
    base = wid * B_PER_W

    pltpu.sync_copy(idx_hbm.at[0, pl.ds(base, B_PER_W)], idx0_v)
    pltpu.sync_copy(idx_hbm.at[1, pl.ds(base, B_PER_W)], idx1_v)
    pltpu.sync_copy(idx_hbm.at[2, pl.ds(base, B_PER_W)], idx2_v)
    pltpu.sync_copy(idx_hbm.at[3, pl.ds(base, B_PER_W)], idx3_v)
    pltpu.sync_copy(wd_hbm, dir_v)
    pltpu.sync_copy(wb_hbm, tab_v.at[pl.ds(0, 128)])
    pltpu.sync_copy(ws_hbm, tab_v.at[pl.ds(128, 256)])
    pltpu.sync_copy(wt_hbm, tab_v.at[pl.ds(384, 128)])

    iota = lax.iota(jnp.int32, NL)
    cols = [iota, iota + 16, iota + 32]

    def repack_row(b, carry):
        bv, d = carry
        for k in range(3):
            idx_ref = (idx0_v, idx1_v, idx2_v)[k]
            tk = plsc.load_gather(idx_ref, [bv]) + ROW_OFF[k]
            for c in range(3):
                v = plsc.load_gather(tab_v, [tk, cols[c]])
                plsc.store_scatter(acc_v, [d + (k * 50 + c * 16)], v)
        return bv + 1, d + D_OUT

    def tail_group(j, dg):
        for k in range(3):
            idx_ref = (idx0_v, idx1_v, idx2_v)[k]
            tkv = idx_ref[pl.ds(j * NL, NL)] + ROW_OFF[k]
            for c in (48, 49):
                v = plsc.load_gather(tab_v, [tkv, jnp.full((NL,), c, jnp.int32)])
                plsc.store_scatter(acc_v, [dg + (k * 50 + c)], v)
        dvals = plsc.load_gather(dir_v, [idx3_v[pl.ds(j * NL, NL)]])
        plsc.store_scatter(acc_v, [dg + 150], dvals)
        return dg + NL * D_OUT

    carry = (jnp.zeros((NL,), jnp.int32), iota)
    carry = lax.fori_loop(0, HALF, repack_row, carry)
    lax.fori_loop(0, HALF // NL, tail_group, iota * D_OUT)
    cp0 = pltpu.async_copy(
        acc_v.at[pl.ds(0, HALF * D_OUT)],
        out_hbm.at[pl.ds(base * D_OUT, HALF * D_OUT)], sem0)

    lax.fori_loop(HALF, B_PER_W, repack_row, carry)
    lax.fori_loop(HALF // NL, B_PER_W // NL, tail_group,
                  (HALF + iota) * D_OUT)
    cp1 = pltpu.async_copy(
        acc_v.at[pl.ds(HALF * D_OUT, HALF * D_OUT)],
        out_hbm.at[pl.ds(base * D_OUT + HALF * D_OUT, HALF * D_OUT)], sem1)
    cp0.wait()
    cp1.wait()


@jax.jit
def _run(idx_t, wb, ws, wt, dir16):
    mesh = plsc.VectorSubcoreMesh(core_axis_name="c", subcore_axis_name="s")
    out_flat = pl.kernel(
        _body,
        out_type=jax.ShapeDtypeStruct((B * D_OUT,), jnp.float32),
        mesh=mesh,
        scratch_types=[
            pltpu.VMEM((B_PER_W,), jnp.int32),
            pltpu.VMEM((B_PER_W,), jnp.int32),
            pltpu.VMEM((B_PER_W,), jnp.int32),
            pltpu.VMEM((B_PER_W,), jnp.int32),
            pltpu.VMEM((NL,), jnp.float32),
            pltpu.VMEM((512, 50), jnp.float32),
            pltpu.VMEM((ACC_WORDS,), jnp.float32),
            pltpu.SemaphoreType.DMA,
            pltpu.SemaphoreType.DMA,
        ],
        compiler_params=pltpu.CompilerParams(
            use_tc_tiling_on_sc=False, needs_layout_passes=False),
    )(idx_t, wb, ws, wt, dir16)
    return out_flat.reshape(B, D_OUT)


def kernel(cat_tensor, W_bus_id, W_station_id, W_time_period, W_direction):
    idx_t = cat_tensor.T.astype(jnp.int32)
    dir16 = jnp.pad(W_direction[:, 0], (0, NL - W_direction.shape[0]))
    return _run(idx_t, W_bus_id, W_station_id, W_time_period, dir16)


# speedup vs baseline: 3.2141x; 1.3171x over previous
import functools

import jax
import jax.numpy as jnp
from jax import lax
from jax.experimental import pallas as pl
from jax.experimental.pallas import tpu as pltpu
from jax.experimental.pallas import tpu_sc as plsc

B = 16384
D_OUT = 151
NC, NS, NL = 2, 16, 16
NW = NC * NS
B_PER_W = B // NW
HALF = B_PER_W // 2
ACC_WORDS = B_PER_W * D_OUT
ROW_OFF = (0, 128, 384)


def _body(idx_hbm, wb_hbm, ws_hbm, wt_hbm, wd_hbm, out_hbm,
          idx0_v, idx1_v, idx2_v, idx3_v, dir_v, tab_v, acc_v,
          sem0, sem1):
    wid = lax.axis_index("s") * NC + lax.axis_index("c")
    base = wid * B_PER_W

    pltpu.sync_copy(idx_hbm.at[0, pl.ds(base, B_PER_W)], idx0_v)
    pltpu.sync_copy(idx_hbm.at[1, pl.ds(base, B_PER_W)], idx1_v)
    pltpu.sync_copy(idx_hbm.at[2, pl.ds(base, B_PER_W)], idx2_v)
    pltpu.sync_copy(idx_hbm.at[3, pl.ds(base, B_PER_W)], idx3_v)
    pltpu.sync_copy(wd_hbm, dir_v)
    pltpu.sync_copy(wb_hbm, tab_v.at[pl.ds(0, 128)])
    pltpu.sync_copy(ws_hbm, tab_v.at[pl.ds(128, 256)])
    pltpu.sync_copy(wt_hbm, tab_v.at[pl.ds(384, 128)])

    iota = lax.iota(jnp.int32, NL)
    cols = [iota, iota + 16, iota + 32]

    def repack_half(lo, hi):
        @plsc.parallel_loop(lo, hi, unroll=4)
        def _rows(b):
            bv = jnp.full((NL,), b, jnp.int32)
            d = jnp.full((NL,), b * D_OUT, jnp.int32) + iota
            for k in range(3):
                idx_ref = (idx0_v, idx1_v, idx2_v)[k]
                tk = plsc.load_gather(idx_ref, [bv]) + ROW_OFF[k]
                for c in range(3):
                    v = plsc.load_gather(tab_v, [tk, cols[c]])
                    plsc.store_scatter(acc_v, [d + (k * 50 + c * 16)], v)

        @plsc.parallel_loop(lo // NL, hi // NL, unroll=2)
        def _tails(j):
            dg = (j * NL + iota) * D_OUT
            for k in range(3):
                idx_ref = (idx0_v, idx1_v, idx2_v)[k]
                tkv = idx_ref[pl.ds(j * NL, NL)] + ROW_OFF[k]
                for c in (48, 49):
                    v = plsc.load_gather(
                        tab_v, [tkv, jnp.full((NL,), c, jnp.int32)])
                    plsc.store_scatter(acc_v, [dg + (k * 50 + c)], v)
            dvals = plsc.load_gather(dir_v, [idx3_v[pl.ds(j * NL, NL)]])
            plsc.store_scatter(acc_v, [dg + 150], dvals)

    repack_half(0, HALF)
    cp0 = pltpu.async_copy(
        acc_v.at[pl.ds(0, HALF * D_OUT)],
        out_hbm.at[pl.ds(base * D_OUT, HALF * D_OUT)], sem0)

    repack_half(HALF, B_PER_W)
    cp1 = pltpu.async_copy(
        acc_v.at[pl.ds(HALF * D_OUT, HALF * D_OUT)],
        out_hbm.at[pl.ds(base * D_OUT + HALF * D_OUT, HALF * D_OUT)], sem1)
    cp0.wait()
    cp1.wait()


@jax.jit
def _run(idx_t, wb, ws, wt, dir16):
    mesh = plsc.VectorSubcoreMesh(core_axis_name="c", subcore_axis_name="s")
    out_flat = pl.kernel(
        _body,
        out_type=jax.ShapeDtypeStruct((B * D_OUT,), jnp.float32),
        mesh=mesh,
        scratch_types=[
            pltpu.VMEM((B_PER_W,), jnp.int32),
            pltpu.VMEM((B_PER_W,), jnp.int32),
            pltpu.VMEM((B_PER_W,), jnp.int32),
            pltpu.VMEM((B_PER_W,), jnp.int32),
            pltpu.VMEM((NL,), jnp.float32),
            pltpu.VMEM((512, 50), jnp.float32),
            pltpu.VMEM((ACC_WORDS,), jnp.float32),
            pltpu.SemaphoreType.DMA,
            pltpu.SemaphoreType.DMA,
        ],
        compiler_params=pltpu.CompilerParams(
            use_tc_tiling_on_sc=False, needs_layout_passes=False),
    )(idx_t, wb, ws, wt, dir16)
    return out_flat.reshape(B, D_OUT)


def kernel(cat_tensor, W_bus_id, W_station_id, W_time_period, W_direction):
    idx_t = cat_tensor.T.astype(jnp.int32)
    dir16 = jnp.pad(W_direction[:, 0], (0, NL - W_direction.shape[0]))
    return _run(idx_t, W_bus_id, W_station_id, W_time_period, dir16)
